# baseline (speedup 1.0000x reference)
import jax
import jax.numpy as jnp
from jax.experimental import pallas as pl
from jax.experimental.pallas import tpu as pltpu

_BM = 500


def _agg_kernel(feature_ref, adj_ref, w_ref, b_ref, out_ref, support_ref):
    @pl.when(pl.program_id(0) == 0)
    def _():
        sup = jnp.dot(feature_ref[...], w_ref[...],
                      preferred_element_type=jnp.float32) + b_ref[...]
        support_ref[...] = sup.astype(jnp.bfloat16)

    a = adj_ref[0].astype(jnp.bfloat16)
    h = jnp.dot(a, support_ref[...], preferred_element_type=jnp.float32)
    out_ref[0] = jnp.tanh(h)


def kernel(feature, adj, W, b):
    n, d = feature.shape
    b2 = b.reshape(1, d)
    adj3 = adj.reshape(n // _BM, _BM, n)
    out3 = pl.pallas_call(
        _agg_kernel,
        grid=(n // _BM,),
        in_specs=[
            pl.BlockSpec((n, d), lambda i: (0, 0)),
            pl.BlockSpec((1, _BM, n), lambda i: (i, 0, 0)),
            pl.BlockSpec((d, d), lambda i: (0, 0)),
            pl.BlockSpec((1, d), lambda i: (0, 0)),
        ],
        out_specs=pl.BlockSpec((1, _BM, d), lambda i: (i, 0, 0)),
        out_shape=jax.ShapeDtypeStruct((n // _BM, _BM, d), jnp.float32),
        scratch_shapes=[pltpu.VMEM((n, d), jnp.bfloat16)],
        compiler_params=pltpu.CompilerParams(
            dimension_semantics=("arbitrary",),
        ),
    )(feature, adj3, W, b2)
    return out3.reshape(n, d)


# speedup vs baseline: 3.5403x; 3.5403x over previous
import jax
import jax.numpy as jnp
from jax.experimental import pallas as pl
from jax.experimental.pallas import tpu as pltpu

_BM = 400


def _agg_kernel(feature_ref, adj_ref, w_ref, b_ref, out_ref, support_ref):
    @pl.when(pl.program_id(0) == 0)
    def _():
        sup = jnp.dot(feature_ref[...], w_ref[...],
                      preferred_element_type=jnp.float32) + b_ref[...]
        support_ref[...] = sup.astype(jnp.bfloat16)

    a = adj_ref[...].astype(jnp.bfloat16)
    h = jnp.dot(a, support_ref[...], preferred_element_type=jnp.float32)
    out_ref[...] = jnp.tanh(h)


def kernel(feature, adj, W, b):
    n, d = feature.shape
    b2 = b.reshape(1, d)
    return pl.pallas_call(
        _agg_kernel,
        grid=(n // _BM,),
        in_specs=[
            pl.BlockSpec((n, d), lambda i: (0, 0)),
            pl.BlockSpec((_BM, n), lambda i: (i, 0)),
            pl.BlockSpec((d, d), lambda i: (0, 0)),
            pl.BlockSpec((1, d), lambda i: (0, 0)),
        ],
        out_specs=pl.BlockSpec((_BM, d), lambda i: (i, 0)),
        out_shape=jax.ShapeDtypeStruct((n, d), jnp.float32),
        scratch_shapes=[pltpu.VMEM((n, d), jnp.bfloat16)],
        compiler_params=pltpu.CompilerParams(
            dimension_semantics=("arbitrary",),
        ),
    )(feature, adj, W, b2)
